# baseline (speedup 1.0000x reference)
import jax
import jax.numpy as jnp
from jax.experimental import pallas as pl
from jax.experimental.pallas import tpu as pltpu
from jax.experimental.pallas import tpu_sc as plsc

EMBED_DIM = 100
PAD_DIM = 128
N_BANKS = 2
LANES = 16
OFFS = (0, 16, 32, 48, 64, 80, EMBED_DIM - LANES)


def _tc_pad_table(table):
    v = table.shape[0]
    blk = 1000 if v % 1000 == 0 else v

    def body(t_ref, o_ref):
        o_ref[...] = jnp.concatenate(
            [t_ref[...], jnp.zeros((blk, PAD_DIM - EMBED_DIM), jnp.float32)],
            axis=1,
        )

    return pl.pallas_call(
        body,
        grid=(v // blk,),
        in_specs=[pl.BlockSpec((blk, EMBED_DIM), lambda i: (i, 0))],
        out_specs=pl.BlockSpec((blk, PAD_DIM), lambda i: (i, 0)),
        out_shape=jax.ShapeDtypeStruct((v, PAD_DIM), jnp.float32),
    )(table)


def _sc_all(iat_pad, pkt_pad, dir_tab, iat_seq, pkt_seq, dir_idx, batch, seq):
    mesh = plsc.VectorSubcoreMesh(core_axis_name="c", subcore_axis_name="s")
    out_struct = jax.ShapeDtypeStruct((batch, seq, EMBED_DIM), jnp.float32)
    n_workers = 32
    rows_per_w = batch // n_workers
    n_iters = rows_per_w // N_BANKS

    @pl.kernel(
        out_type=(out_struct, out_struct, out_struct),
        mesh=mesh,
        scratch_types=(
            [pltpu.VMEM((1, seq), jnp.int32)] * 3
            + [pltpu.VMEM((seq, PAD_DIM), jnp.float32)] * 3
            + [pltpu.VMEM((seq, EMBED_DIM), jnp.float32)] * 3
        ) * N_BANKS
        + [pltpu.SemaphoreType.DMA] * (3 * N_BANKS),
    )
    def k(iat_t, pkt_t, dir_t, ii_h, pi_h, di_h, io_h, po_h, do_h,
          ii0, pi0, di0, gi0, gp0, gd0, ni0, np0, nd0,
          ii1, pi1, di1, gi1, gp1, gd1, ni1, np1, nd1,
          isem0, isem1, gsem0, gsem1, wsem0, wsem1):
        wid = jax.lax.axis_index("s") * 2 + jax.lax.axis_index("c")
        base = wid * rows_per_w
        ii, pi, di = (ii0, ii1), (pi0, pi1), (di0, di1)
        g_i, g_p, g_d = (gi0, gi1), (gp0, gp1), (gd0, gd1)
        n_i, n_p, n_d = (ni0, ni1), (np0, np1), (nd0, nd1)
        isems = (isem0, isem1)
        gsems = (gsem0, gsem1)
        wsems = (wsem0, wsem1)

        def idx_copies(bank, step):
            sl = pl.ds(base + step, 1)
            return [
                pltpu.make_async_copy(ii_h.at[sl], ii[bank], isems[bank]),
                pltpu.make_async_copy(pi_h.at[sl], pi[bank], isems[bank]),
                pltpu.make_async_copy(di_h.at[sl], di[bank], isems[bank]),
            ]

        def gather_copies(bank):
            return [
                pltpu.make_async_copy(
                    iat_t.at[ii[bank].at[0]], g_i[bank], gsems[bank]),
                pltpu.make_async_copy(
                    pkt_t.at[pi[bank].at[0]], g_p[bank], gsems[bank]),
                pltpu.make_async_copy(
                    dir_t.at[di[bank].at[0]], g_d[bank], gsems[bank]),
            ]

        def write_copies(bank, step):
            r = base + step
            return [
                pltpu.make_async_copy(n_i[bank], io_h.at[r], wsems[bank]),
                pltpu.make_async_copy(n_p[bank], po_h.at[r], wsems[bank]),
                pltpu.make_async_copy(n_d[bank], do_h.at[r], wsems[bank]),
            ]

        def start(copies):
            for h in copies:
                h.start()

        def wait(copies):
            for h in copies:
                h.wait()

        def narrow(bank):
            @pl.loop(0, seq)
            def _(r):
                for off in OFFS:
                    sl = pl.ds(off, LANES)
                    n_i[bank][r, sl] = g_i[bank][r, sl]
                    n_p[bank][r, sl] = g_p[bank][r, sl]
                    n_d[bank][r, sl] = g_d[bank][r, sl]

        start(idx_copies(0, 0))
        start(idx_copies(1, 1))
        wait(idx_copies(0, 0))
        start(gather_copies(0))

        @pl.loop(0, n_iters)
        def _(i):
            s0 = 2 * i
            s1 = 2 * i + 1

            wait(idx_copies(1, s1))
            start(gather_copies(1))

            wait(gather_copies(0))

            @pl.when(i > 0)
            def _():
                wait(write_copies(0, 0))

            narrow(0)
            start(write_copies(0, s0))

            @pl.when(i < n_iters - 1)
            def _():
                start(idx_copies(0, s0 + 2))

            wait(gather_copies(1))

            @pl.when(i > 0)
            def _():
                wait(write_copies(1, 0))

            narrow(1)
            start(write_copies(1, s1))

            @pl.when(i < n_iters - 1)
            def _():
                start(idx_copies(1, s1 + 2))
                wait(idx_copies(0, s0 + 2))
                start(gather_copies(0))

        wait(write_copies(0, 0))
        wait(write_copies(1, 0))

    return k(iat_pad, pkt_pad, dir_tab, iat_seq, pkt_seq, dir_idx)


def kernel(pkt_len_seq, pkt_dir_seq, iat_seq, pkt_len_table, iat_table):
    batch, seq = pkt_len_seq.shape

    iat_pad = _tc_pad_table(iat_table)
    pkt_pad = _tc_pad_table(pkt_len_table)
    dir_tab = jnp.stack([
        jnp.full((PAD_DIM,), -1.0, jnp.float32),
        jnp.full((PAD_DIM,), 1.0, jnp.float32),
    ])
    dir_idx = ((pkt_dir_seq.astype(jnp.int32) + 1) >> 1).astype(jnp.int32)

    iat_out, pkt_out, dir_out = _sc_all(
        iat_pad, pkt_pad, dir_tab,
        iat_seq.astype(jnp.int32), pkt_len_seq.astype(jnp.int32), dir_idx,
        batch, seq,
    )
    return (pkt_out, dir_out, iat_out)


# speedup vs baseline: 4.5558x; 4.5558x over previous
import jax
import jax.numpy as jnp
from jax.experimental import pallas as pl
from jax.experimental.pallas import tpu as pltpu
from jax.experimental.pallas import tpu_sc as plsc

EMBED_DIM = 100
PAD_DIM = 128
B_BLK = 2
LANES = 16
OFFS = (0, 16, 32, 48, 64, 80, EMBED_DIM - LANES)
DIR_REP = 128


def _tc_pad_table(table):
    v = table.shape[0]
    blk = 1000 if v % 1000 == 0 else v

    def body(t_ref, o_ref):
        o_ref[...] = jnp.concatenate(
            [t_ref[...], jnp.zeros((blk, PAD_DIM - EMBED_DIM), jnp.float32)],
            axis=1,
        )

    return pl.pallas_call(
        body,
        grid=(v // blk,),
        in_specs=[pl.BlockSpec((blk, EMBED_DIM), lambda i: (i, 0))],
        out_specs=pl.BlockSpec((blk, PAD_DIM), lambda i: (i, 0)),
        out_shape=jax.ShapeDtypeStruct((v, PAD_DIM), jnp.float32),
    )(table)


def _sc_all(iat_pad, pkt_pad, dir_tab, iat_seq, pkt_seq, dir_idx, batch, seq):
    mesh = plsc.VectorSubcoreMesh(core_axis_name="c", subcore_axis_name="s")
    out_struct = jax.ShapeDtypeStruct((batch, seq, EMBED_DIM), jnp.float32)

    @pl.kernel(
        out_type=(out_struct, out_struct, out_struct),
        mesh=mesh,
        scratch_types=[
            pltpu.VMEM((B_BLK, seq, PAD_DIM), jnp.float32),
            pltpu.VMEM((B_BLK, seq, PAD_DIM), jnp.float32),
            pltpu.VMEM((B_BLK, seq, PAD_DIM), jnp.float32),
            pltpu.SemaphoreType.DMA,
        ],
    )
    def k(iat_t, pkt_t, dir_t, ii_h, pi_h, di_h, io_h, po_h, do_h,
          ig_v, pg_v, dg_v, gsem):
        def body(ii_vmem, pi_vmem, di_vmem, io_vmem, po_vmem, do_vmem):
            gathers = []
            for j in range(B_BLK):
                gathers.append(pltpu.async_copy(
                    iat_t.at[ii_vmem.at[j]], ig_v.at[j], gsem))
                gathers.append(pltpu.async_copy(
                    pkt_t.at[pi_vmem.at[j]], pg_v.at[j], gsem))
                gathers.append(pltpu.async_copy(
                    dir_t.at[di_vmem.at[j]], dg_v.at[j], gsem))
            for g in gathers:
                g.wait()

            @pl.loop(0, seq)
            def _(r):
                for j in range(B_BLK):
                    for off in OFFS:
                        sl = pl.ds(off, LANES)
                        io_vmem[j, r, sl] = ig_v[j, r, sl]
                        po_vmem[j, r, sl] = pg_v[j, r, sl]
                        do_vmem[j, r, sl] = dg_v[j, r, sl]

        pltpu.emit_pipeline(
            body,
            grid=(batch // B_BLK,),
            in_specs=[
                pl.BlockSpec((B_BLK, seq), lambda i: (i, 0)),
                pl.BlockSpec((B_BLK, seq), lambda i: (i, 0)),
                pl.BlockSpec((B_BLK, seq), lambda i: (i, 0)),
            ],
            out_specs=[
                pl.BlockSpec((B_BLK, seq, EMBED_DIM), lambda i: (i, 0, 0)),
                pl.BlockSpec((B_BLK, seq, EMBED_DIM), lambda i: (i, 0, 0)),
                pl.BlockSpec((B_BLK, seq, EMBED_DIM), lambda i: (i, 0, 0)),
            ],
            core_axis_name=("c", "s"),
            dimension_semantics=(pltpu.PARALLEL,),
        )(ii_h, pi_h, di_h, io_h, po_h, do_h)

    return k(iat_pad, pkt_pad, dir_tab, iat_seq, pkt_seq, dir_idx)


def kernel(pkt_len_seq, pkt_dir_seq, iat_seq, pkt_len_table, iat_table):
    batch, seq = pkt_len_seq.shape

    iat_pad = _tc_pad_table(iat_table)
    pkt_pad = _tc_pad_table(pkt_len_table)
    dir_tab = jnp.concatenate([
        jnp.full((DIR_REP, PAD_DIM), -1.0, jnp.float32),
        jnp.full((DIR_REP, PAD_DIM), 1.0, jnp.float32),
    ])
    stripe = jnp.broadcast_to(
        jnp.arange(seq, dtype=jnp.int32)[None, :] % DIR_REP, (batch, seq))
    dir_bit = (pkt_dir_seq.astype(jnp.int32) + 1) >> 1
    dir_idx = dir_bit * DIR_REP + stripe

    iat_out, pkt_out, dir_out = _sc_all(
        iat_pad, pkt_pad, dir_tab,
        iat_seq.astype(jnp.int32), pkt_len_seq.astype(jnp.int32), dir_idx,
        batch, seq,
    )
    return (pkt_out, dir_out, iat_out)


# speedup vs baseline: 5.7351x; 1.2589x over previous
import jax
import jax.numpy as jnp
from jax.experimental import pallas as pl
from jax.experimental.pallas import tpu as pltpu
from jax.experimental.pallas import tpu_sc as plsc

EMBED_DIM = 100
PAD_DIM = 128
B_BLK = 4
WAVE = 2
LANES = 16
OFFS = (0, 16, 32, 48, 64, 80, EMBED_DIM - LANES)
DIR_REP = 128


def _tc_pad_table(table):
    v = table.shape[0]
    blk = 1000 if v % 1000 == 0 else v

    def body(t_ref, o_ref):
        o_ref[...] = jnp.concatenate(
            [t_ref[...], jnp.zeros((blk, PAD_DIM - EMBED_DIM), jnp.float32)],
            axis=1,
        )

    return pl.pallas_call(
        body,
        grid=(v // blk,),
        in_specs=[pl.BlockSpec((blk, EMBED_DIM), lambda i: (i, 0))],
        out_specs=pl.BlockSpec((blk, PAD_DIM), lambda i: (i, 0)),
        out_shape=jax.ShapeDtypeStruct((v, PAD_DIM), jnp.float32),
    )(table)


def _sc_all(iat_pad, pkt_pad, dir_tab, iat_seq, pkt_seq, dir_idx, sid_arr,
            batch, seq):
    mesh = plsc.VectorSubcoreMesh(core_axis_name="c", subcore_axis_name="s")
    out_struct = jax.ShapeDtypeStruct((batch, seq, EMBED_DIM), jnp.float32)

    @pl.kernel(
        out_type=(out_struct, out_struct, out_struct),
        mesh=mesh,
        scratch_types=[
            pltpu.VMEM((B_BLK, seq, PAD_DIM), jnp.float32),
            pltpu.VMEM((B_BLK, seq, PAD_DIM), jnp.float32),
            pltpu.VMEM((WAVE, seq, PAD_DIM), jnp.float32),
            pltpu.VMEM((WAVE, seq, EMBED_DIM), jnp.float32),
            pltpu.VMEM((WAVE, seq, EMBED_DIM), jnp.float32),
            pltpu.VMEM((WAVE, seq, EMBED_DIM), jnp.float32),
            pltpu.SemaphoreType.DMA,
            pltpu.SemaphoreType.DMA,
            pltpu.SemaphoreType.DMA,
        ],
    )
    def k(iat_t, pkt_t, dir_t, ii_h, pi_h, di_h, sid_h, io_h, po_h, do_h,
          g_i, g_p, g_d, n_i_s, n_p_s, n_d_s,
          gsemA, gsemB, wsem):
        outs = (io_h, po_h, do_h)

        def wave_writes(bufs, sem, b0, j0):
            return [
                pltpu.make_async_copy(buf.at[jj], out.at[b0 + j0 + jj], sem)
                for buf, out in zip(bufs, outs)
                for jj in range(WAVE)
            ]

        def prime_writes(bufs, sem):
            for buf, out in zip(bufs, outs):
                for jj in range(WAVE):
                    pltpu.make_async_copy(out.at[0], buf.at[jj], sem).start()

        def narrow(bufs, j0):
            n_i, n_p, n_d = bufs

            @pl.loop(0, seq)
            def _(r):
                for jj in range(WAVE):
                    j = j0 + jj
                    for off in OFFS:
                        sl = pl.ds(off, LANES)
                        n_i[jj, r, sl] = g_i[j, r, sl]
                        n_p[jj, r, sl] = g_p[j, r, sl]
                        n_d[jj, r, sl] = g_d[jj, r, sl]

        nn = (n_i_s, n_p_s, n_d_s)

        prime_writes(nn, wsem)

        def body(ii_vmem, pi_vmem, di_vmem, sid_vmem):
            sid_row = sid_vmem.at[0][...]
            sid = jax.lax.squeeze(jax.lax.slice(sid_row, (0,), (1,)), (0,))
            b0 = sid * B_BLK

            ga, gb = [], []
            for j in range(B_BLK):
                sem = gsemA if j < WAVE else gsemB
                dst = ga if j < WAVE else gb
                dst.append(pltpu.async_copy(
                    iat_t.at[ii_vmem.at[j]], g_i.at[j], sem))
                dst.append(pltpu.async_copy(
                    pkt_t.at[pi_vmem.at[j]], g_p.at[j], sem))
            for jj in range(WAVE):
                ga.append(pltpu.async_copy(
                    dir_t.at[di_vmem.at[jj]], g_d.at[jj], gsemA))

            for g in ga:
                g.wait()
            for h in wave_writes(nn, wsem, 0, 0):
                h.wait()
            narrow(nn, 0)
            for h in wave_writes(nn, wsem, b0, 0):
                h.start()

            gd2 = [pltpu.async_copy(
                dir_t.at[di_vmem.at[WAVE + jj]], g_d.at[jj], gsemB)
                for jj in range(WAVE)]
            for g in gb + gd2:
                g.wait()
            for h in wave_writes(nn, wsem, 0, 0):
                h.wait()
            narrow(nn, WAVE)
            for h in wave_writes(nn, wsem, b0, WAVE):
                h.start()

        pltpu.emit_pipeline(
            body,
            grid=(batch // B_BLK,),
            in_specs=[
                pl.BlockSpec((B_BLK, seq), lambda i: (i, 0)),
                pl.BlockSpec((B_BLK, seq), lambda i: (i, 0)),
                pl.BlockSpec((B_BLK, seq), lambda i: (i, 0)),
                pl.BlockSpec((1, 16), lambda i: (i, 0)),
            ],
            core_axis_name=("c", "s"),
            dimension_semantics=(pltpu.PARALLEL,),
        )(ii_h, pi_h, di_h, sid_h)

        for h in wave_writes(nn, wsem, 0, 0):
            h.wait()

    return k(iat_pad, pkt_pad, dir_tab, iat_seq, pkt_seq, dir_idx, sid_arr)


def kernel(pkt_len_seq, pkt_dir_seq, iat_seq, pkt_len_table, iat_table):
    batch, seq = pkt_len_seq.shape

    iat_pad = _tc_pad_table(iat_table)
    pkt_pad = _tc_pad_table(pkt_len_table)
    dir_tab = jnp.concatenate([
        jnp.full((DIR_REP, PAD_DIM), -1.0, jnp.float32),
        jnp.full((DIR_REP, PAD_DIM), 1.0, jnp.float32),
    ])
    stripe = jnp.broadcast_to(
        jnp.arange(seq, dtype=jnp.int32)[None, :] % DIR_REP, (batch, seq))
    dir_bit = (pkt_dir_seq.astype(jnp.int32) + 1) >> 1
    dir_idx = dir_bit * DIR_REP + stripe
    sid_arr = jnp.broadcast_to(
        jnp.arange(batch // B_BLK, dtype=jnp.int32)[:, None],
        (batch // B_BLK, 16))

    iat_out, pkt_out, dir_out = _sc_all(
        iat_pad, pkt_pad, dir_tab,
        iat_seq.astype(jnp.int32), pkt_len_seq.astype(jnp.int32), dir_idx,
        sid_arr, batch, seq,
    )
    return (pkt_out, dir_out, iat_out)
